# Initial kernel scaffold; baseline (speedup 1.0000x reference)
import jax
import jax.numpy as jnp
from jax.experimental import pallas as pl


def kernel(batch_x, emb_table, W1, b1, W2, b2):
    raise NotImplementedError("write your pallas kernel here")


# speedup vs baseline: 5.1244x; 5.1244x over previous
import functools

import jax
import jax.numpy as jnp
from jax import lax
from jax.experimental import pallas as pl
from jax.experimental.pallas import tpu as pltpu
from jax.experimental.pallas import tpu_sc as plsc

VOCAB = 38
SEQ = 64
EMB = 128
HID = 256
BATCH = 4096

VPAD = 64
NROWS = SEQ * VPAD
L = 16
NC = 2
NS = 16
NW = NC * NS
BPW = BATCH // NW


def _table_body(emb_ref, w1_ref, b1_ref, out_ref):
    s = pl.program_id(0)
    m = jnp.dot(emb_ref[...], w1_ref[0], preferred_element_type=jnp.float32)
    m = m + b1_ref[...] * (s == 0).astype(jnp.float32)
    out_ref[...] = m


def _build_table(emb_pad, w1r, b1):
    return pl.pallas_call(
        _table_body,
        grid=(SEQ,),
        in_specs=[
            pl.BlockSpec((VPAD, EMB), lambda s: (0, 0)),
            pl.BlockSpec((1, EMB, HID), lambda s: (s, 0, 0)),
            pl.BlockSpec((1, HID), lambda s: (0, 0)),
        ],
        out_specs=pl.BlockSpec((VPAD, HID), lambda s: (s, 0)),
        out_shape=jax.ShapeDtypeStruct((NROWS, HID), jnp.float32),
    )(emb_pad, w1r, b1.reshape(1, HID))


@functools.cache
def _make_sc_forward():
    mesh = plsc.VectorSubcoreMesh(core_axis_name="c", subcore_axis_name="s")
    return functools.partial(
        pl.kernel,
        mesh=mesh,
        out_type=jax.ShapeDtypeStruct((BATCH,), jnp.float32),
        scratch_types=[
            pltpu.VMEM((BPW * SEQ,), jnp.int32),
            pltpu.VMEM((2, SEQ), jnp.int32),
            pltpu.VMEM((2, SEQ, HID), jnp.float32),
            pltpu.VMEM((HID,), jnp.float32),
            pltpu.VMEM((L,), jnp.float32),
            pltpu.VMEM((BPW,), jnp.float32),
            pltpu.SemaphoreType.DMA,
            pltpu.SemaphoreType.DMA,
        ],
    )(_sc_body)


_GDN = lax.GatherDimensionNumbers(
    offset_dims=(), collapsed_slice_dims=(0,), start_index_map=(0,))


def _lane_perm(v, idx):
    return lax.gather(v, idx[:, None], _GDN, slice_sizes=(1,),
                      mode=lax.GatherScatterMode.PROMISE_IN_BOUNDS)


def _lane_sum(v):
    lanes = lax.iota(jnp.int32, L)
    for k in (1, 2, 4, 8):
        v = v + _lane_perm(v, lanes ^ k)
    return v


def _sc_body(x_hbm, m_hbm, w2_hbm, b2_hbm, out_hbm,
                xs_v, idx_v, rows_v, w2_v, b2_v, log_v, sem0, sem1):
    wid = lax.axis_index("s") * NC + lax.axis_index("c")
    base = wid * BPW

    pltpu.sync_copy(x_hbm.at[pl.ds(base * SEQ, BPW * SEQ)], xs_v)
    pltpu.sync_copy(w2_hbm, w2_v)
    pltpu.sync_copy(b2_hbm, b2_v)

    sems = (sem0, sem1)
    offs = [(lax.iota(jnp.int32, L) + k * L) * VPAD for k in range(SEQ // L)]

    def compute_idx(i, b):
        for k in range(SEQ // L):
            xv = xs_v[pl.ds(i * SEQ + k * L, L)]
            idx_v[b, pl.ds(k * L, L)] = xv + offs[k]

    def start(b):
        pltpu.async_copy(m_hbm.at[idx_v.at[b]], rows_v.at[b], sems[b])

    def wait(b):
        pltpu.make_async_copy(m_hbm.at[idx_v.at[b]], rows_v.at[b], sems[b]).wait()

    lanes = lax.iota(jnp.int32, L)

    def accumulate(i, b, logacc):
        def chunk_body(c, dot):
            co = c * L
            acc = rows_v[b, 0, pl.ds(co, L)]
            for j in range(1, SEQ):
                acc = acc + rows_v[b, j, pl.ds(co, L)]
            h = jnp.maximum(acc, 0.0)
            return dot + h * w2_v[pl.ds(co, L)]

        dot = lax.fori_loop(0, HID // L, chunk_body, b2_v[...])
        tot = _lane_sum(dot)
        return logacc + jnp.where(lanes == (i & (L - 1)), tot, 0.0)

    compute_idx(0, 0)
    start(0)

    def pair_body(p, logacc):
        i0 = 2 * p
        compute_idx(i0 + 1, 1)
        start(1)
        wait(0)
        logacc = accumulate(i0, 0, logacc)

        @pl.when(p < BPW // 2 - 1)
        def _():
            compute_idx(i0 + 2, 0)
            start(0)

        wait(1)
        logacc = accumulate(i0 + 1, 1, logacc)

        group_done = (p & 7) == 7

        @pl.when(group_done)
        def _():
            log_v[pl.ds((p >> 3) * L, L)] = logacc

        return jnp.where(group_done, 0.0, logacc)

    lax.fori_loop(0, BPW // 2, pair_body, jnp.zeros((L,), jnp.float32))

    pltpu.sync_copy(log_v, out_hbm.at[pl.ds(base, BPW)])


def kernel(batch_x, emb_table, W1, b1, W2, b2):
    emb_pad = jnp.zeros((VPAD, EMB), jnp.float32).at[:VOCAB].set(emb_table)
    w1r = W1.reshape(SEQ, EMB, HID)
    table = _build_table(emb_pad, w1r, b1)

    x_flat = batch_x.astype(jnp.int32).reshape(-1)
    w2 = W2.reshape(HID)
    b2_pad = jnp.zeros((L,), jnp.float32).at[0].set(b2[0])
    return _make_sc_forward()(x_flat, table, w2, b2_pad)


# speedup vs baseline: 5.4493x; 1.0634x over previous
import functools

import jax
import jax.numpy as jnp
from jax import lax
from jax.experimental import pallas as pl
from jax.experimental.pallas import tpu as pltpu
from jax.experimental.pallas import tpu_sc as plsc

VOCAB = 38
SEQ = 64
EMB = 128
HID = 256
BATCH = 4096

NPAIR = SEQ // 2
PROWS = VOCAB * VOCAB
NROWS = NPAIR * PROWS
L = 16
NC = 2
NS = 16
NW = NC * NS
BPW = BATCH // NW


def _table_body(emb_ref, w1a_ref, w1b_ref, b1_ref, out_ref):
    s = pl.program_id(0)
    m1 = jnp.dot(emb_ref[...], w1a_ref[0], preferred_element_type=jnp.float32)
    m2 = jnp.dot(emb_ref[...], w1b_ref[0], preferred_element_type=jnp.float32)
    m1 = m1 + b1_ref[...] * (s == 0).astype(jnp.float32)
    out_ref[0] = m1[:, None, :] + m2[None, :, :]


def _build_table(emb, w1r, b1):
    return pl.pallas_call(
        _table_body,
        grid=(NPAIR,),
        in_specs=[
            pl.BlockSpec((VOCAB, EMB), lambda s: (0, 0)),
            pl.BlockSpec((1, EMB, HID), lambda s: (s, 0, 0)),
            pl.BlockSpec((1, EMB, HID), lambda s: (s + NPAIR, 0, 0)),
            pl.BlockSpec((1, HID), lambda s: (0, 0)),
        ],
        out_specs=pl.BlockSpec((1, VOCAB, VOCAB, HID), lambda s: (s, 0, 0, 0)),
        out_shape=jax.ShapeDtypeStruct((NPAIR, VOCAB, VOCAB, HID), jnp.float32),
    )(emb, w1r, w1r, b1.reshape(1, HID))


@functools.cache
def _make_sc_forward():
    mesh = plsc.VectorSubcoreMesh(core_axis_name="c", subcore_axis_name="s")
    return functools.partial(
        pl.kernel,
        mesh=mesh,
        out_type=jax.ShapeDtypeStruct((BATCH,), jnp.float32),
        scratch_types=[
            pltpu.VMEM((BPW * SEQ,), jnp.int32),
            pltpu.VMEM((2, NPAIR), jnp.int32),
            pltpu.VMEM((2, NPAIR, HID), jnp.float32),
            pltpu.VMEM((HID,), jnp.float32),
            pltpu.VMEM((L,), jnp.float32),
            pltpu.VMEM((BPW,), jnp.float32),
            pltpu.SemaphoreType.DMA,
            pltpu.SemaphoreType.DMA,
        ],
    )(_sc_body)


_GDN = lax.GatherDimensionNumbers(
    offset_dims=(), collapsed_slice_dims=(0,), start_index_map=(0,))


def _lane_perm(v, idx):
    return lax.gather(v, idx[:, None], _GDN, slice_sizes=(1,),
                      mode=lax.GatherScatterMode.PROMISE_IN_BOUNDS)


def _lane_sum(v):
    lanes = lax.iota(jnp.int32, L)
    for k in (1, 2, 4, 8):
        v = v + _lane_perm(v, lanes ^ k)
    return v


def _sc_body(x_hbm, t2_hbm, w2_hbm, b2_hbm, out_hbm,
             xs_v, idx_v, rows_v, w2_v, b2_v, log_v, sem0, sem1):
    wid = lax.axis_index("s") * NC + lax.axis_index("c")
    base = wid * BPW

    pltpu.sync_copy(x_hbm.at[pl.ds(base * SEQ, BPW * SEQ)], xs_v)
    pltpu.sync_copy(w2_hbm, w2_v)
    pltpu.sync_copy(b2_hbm, b2_v)

    sems = (sem0, sem1)
    offs = [(lax.iota(jnp.int32, L) + k * L) * PROWS for k in range(NPAIR // L)]

    def compute_idx(i, b):
        xv = [xs_v[pl.ds(i * SEQ + k * L, L)] for k in range(SEQ // L)]
        for k in range(NPAIR // L):
            idx_v[b, pl.ds(k * L, L)] = (
                offs[k] + xv[k] * VOCAB + xv[k + NPAIR // L])

    def start(b):
        pltpu.async_copy(t2_hbm.at[idx_v.at[b]], rows_v.at[b], sems[b])

    def wait(b):
        pltpu.make_async_copy(t2_hbm.at[idx_v.at[b]], rows_v.at[b],
                              sems[b]).wait()

    lanes = lax.iota(jnp.int32, L)

    def accumulate(i, b, logacc):
        def chunk_body(c, dot):
            co = c * L
            acc = rows_v[b, 0, pl.ds(co, L)]
            for j in range(1, NPAIR):
                acc = acc + rows_v[b, j, pl.ds(co, L)]
            h = jnp.maximum(acc, 0.0)
            return dot + h * w2_v[pl.ds(co, L)]

        dot = lax.fori_loop(0, HID // L, chunk_body, b2_v[...])
        tot = _lane_sum(dot)
        return logacc + jnp.where(lanes == (i & (L - 1)), tot, 0.0)

    compute_idx(0, 0)
    start(0)

    def pair_body(p, logacc):
        i0 = 2 * p
        compute_idx(i0 + 1, 1)
        start(1)
        wait(0)
        logacc = accumulate(i0, 0, logacc)

        @pl.when(p < BPW // 2 - 1)
        def _():
            compute_idx(i0 + 2, 0)
            start(0)

        wait(1)
        logacc = accumulate(i0 + 1, 1, logacc)

        group_done = (p & 7) == 7

        @pl.when(group_done)
        def _():
            log_v[pl.ds((p >> 3) * L, L)] = logacc

        return jnp.where(group_done, 0.0, logacc)

    lax.fori_loop(0, BPW // 2, pair_body, jnp.zeros((L,), jnp.float32))

    pltpu.sync_copy(log_v, out_hbm.at[pl.ds(base, BPW)])


def kernel(batch_x, emb_table, W1, b1, W2, b2):
    w1r = W1.reshape(SEQ, EMB, HID)
    table = _build_table(emb_table, w1r, b1).reshape(NROWS, HID)

    x_flat = batch_x.astype(jnp.int32).reshape(-1)
    w2 = W2.reshape(HID)
    b2_pad = jnp.zeros((L,), jnp.float32).at[0].set(b2[0])
    return _make_sc_forward()(x_flat, table, w2, b2_pad)


# speedup vs baseline: 6.9942x; 1.2835x over previous
import functools

import jax
import jax.numpy as jnp
from jax import lax
from jax.experimental import pallas as pl
from jax.experimental.pallas import tpu as pltpu
from jax.experimental.pallas import tpu_sc as plsc

VOCAB = 38
SEQ = 64
EMB = 128
HID = 256
BATCH = 4096

NPAIR = SEQ // 2
VPAD = 40
PROWS = VPAD * VPAD
NROWS = NPAIR * PROWS
L = 16
NC = 2
NS = 16
NW = NC * NS
BPW = BATCH // NW


def _table_body(emb_ref, w1a_ref, w1b_ref, b1_ref, out_ref):
    s = pl.program_id(0)
    m1 = jnp.dot(emb_ref[...], w1a_ref[0], preferred_element_type=jnp.float32)
    m2 = jnp.dot(emb_ref[...], w1b_ref[0], preferred_element_type=jnp.float32)
    m1 = m1 + b1_ref[...] * (s == 0).astype(jnp.float32)
    out_ref[0] = m1[:, None, :] + m2[None, :, :]


def _build_table(emb, w1r, b1):
    return pl.pallas_call(
        _table_body,
        grid=(NPAIR,),
        in_specs=[
            pl.BlockSpec((VPAD, EMB), lambda s: (0, 0)),
            pl.BlockSpec((1, EMB, HID), lambda s: (s, 0, 0)),
            pl.BlockSpec((1, EMB, HID), lambda s: (s + NPAIR, 0, 0)),
            pl.BlockSpec((1, HID), lambda s: (0, 0)),
        ],
        out_specs=pl.BlockSpec((1, VPAD, VPAD, HID), lambda s: (s, 0, 0, 0)),
        out_shape=jax.ShapeDtypeStruct((NPAIR, VPAD, VPAD, HID), jnp.float32),
    )(emb, w1r, w1r, b1.reshape(1, HID))


@functools.cache
def _make_sc_forward():
    mesh = plsc.VectorSubcoreMesh(core_axis_name="c", subcore_axis_name="s")
    return functools.partial(
        pl.kernel,
        mesh=mesh,
        out_type=jax.ShapeDtypeStruct((BATCH,), jnp.float32),
        scratch_types=[
            pltpu.VMEM((BPW * SEQ,), jnp.int32),
            pltpu.VMEM((2, NPAIR), jnp.int32),
            pltpu.VMEM((2, NPAIR, HID), jnp.float32),
            pltpu.VMEM((HID,), jnp.float32),
            pltpu.VMEM((L,), jnp.float32),
            pltpu.VMEM((BPW,), jnp.float32),
            pltpu.SemaphoreType.DMA,
            pltpu.SemaphoreType.DMA,
        ],
    )(_sc_body)


_GDN = lax.GatherDimensionNumbers(
    offset_dims=(), collapsed_slice_dims=(0,), start_index_map=(0,))


def _lane_perm(v, idx):
    return lax.gather(v, idx[:, None], _GDN, slice_sizes=(1,),
                      mode=lax.GatherScatterMode.PROMISE_IN_BOUNDS)


def _lane_sum(v):
    lanes = lax.iota(jnp.int32, L)
    for k in (1, 2, 4, 8):
        v = v + _lane_perm(v, lanes ^ k)
    return v


def _sc_body(x_hbm, t2_hbm, w2_hbm, b2_hbm, out_hbm,
             xs_v, idx_v, rows_v, w2_v, b2_v, log_v, sem0, sem1):
    wid = lax.axis_index("s") * NC + lax.axis_index("c")
    base = wid * BPW

    pltpu.sync_copy(x_hbm.at[pl.ds(base * SEQ, BPW * SEQ)], xs_v)
    pltpu.sync_copy(w2_hbm, w2_v)
    pltpu.sync_copy(b2_hbm, b2_v)

    sems = (sem0, sem1)
    offs = [(lax.iota(jnp.int32, L) + k * L) * PROWS for k in range(NPAIR // L)]

    def compute_idx(i, b):
        xv = [xs_v[pl.ds(i * SEQ + k * L, L)] for k in range(SEQ // L)]
        for k in range(NPAIR // L):
            idx_v[b, pl.ds(k * L, L)] = (
                offs[k] + xv[k] * VPAD + xv[k + NPAIR // L])

    def start(b):
        pltpu.async_copy(t2_hbm.at[idx_v.at[b]], rows_v.at[b], sems[b])

    def wait(b):
        pltpu.make_async_copy(t2_hbm.at[idx_v.at[b]], rows_v.at[b],
                              sems[b]).wait()

    lanes = lax.iota(jnp.int32, L)

    def accumulate(i, b, logacc):
        def chunk_body(c, dot):
            co = c * L
            acc = rows_v[b, 0, pl.ds(co, L)]
            for j in range(1, NPAIR):
                acc = acc + rows_v[b, j, pl.ds(co, L)]
            h = jnp.maximum(acc, 0.0)
            return dot + h * w2_v[pl.ds(co, L)]

        dot = lax.fori_loop(0, HID // L, chunk_body, b2_v[...])
        tot = _lane_sum(dot)
        return logacc + jnp.where(lanes == (i & (L - 1)), tot, 0.0)

    compute_idx(0, 0)
    start(0)

    def pair_body(p, logacc):
        i0 = 2 * p
        compute_idx(i0 + 1, 1)
        start(1)
        wait(0)
        logacc = accumulate(i0, 0, logacc)

        @pl.when(p < BPW // 2 - 1)
        def _():
            compute_idx(i0 + 2, 0)
            start(0)

        wait(1)
        logacc = accumulate(i0 + 1, 1, logacc)

        group_done = (p & 7) == 7

        @pl.when(group_done)
        def _():
            log_v[pl.ds((p >> 3) * L, L)] = logacc

        return jnp.where(group_done, 0.0, logacc)

    lax.fori_loop(0, BPW // 2, pair_body, jnp.zeros((L,), jnp.float32))

    pltpu.sync_copy(log_v, out_hbm.at[pl.ds(base, BPW)])


def kernel(batch_x, emb_table, W1, b1, W2, b2):
    w1r = W1.reshape(SEQ, EMB, HID)
    emb_pad = jnp.zeros((VPAD, EMB), jnp.float32).at[:VOCAB].set(emb_table)
    table = _build_table(emb_pad, w1r, b1).reshape(NROWS, HID)

    x_flat = batch_x.astype(jnp.int32).reshape(-1)
    w2 = W2.reshape(HID)
    b2_pad = jnp.zeros((L,), jnp.float32).at[0].set(b2[0])
    return _make_sc_forward()(x_flat, table, w2, b2_pad)


# speedup vs baseline: 8.7505x; 1.2511x over previous
import functools

import jax
import jax.numpy as jnp
from jax import lax
from jax.experimental import pallas as pl
from jax.experimental.pallas import tpu as pltpu
from jax.experimental.pallas import tpu_sc as plsc

VOCAB = 38
SEQ = 64
EMB = 128
HID = 256
BATCH = 4096

NPAIR = SEQ // 2
VPAD = 40
PROWS = VPAD * VPAD
NROWS = NPAIR * PROWS
L = 16
NC = 2
NS = 16
NW = NC * NS
BPW = BATCH // NW


def _table_body(emb_ref, w1a_ref, w1b_ref, b1_ref, out_ref):
    s = pl.program_id(0)
    m1 = jnp.dot(emb_ref[...], w1a_ref[0], preferred_element_type=jnp.float32)
    m2 = jnp.dot(emb_ref[...], w1b_ref[0], preferred_element_type=jnp.float32)
    m1 = m1 + b1_ref[...] * (s == 0).astype(jnp.float32)
    out_ref[0] = m1[:, None, :] + m2[None, :, :]


def _build_table(emb, w1r, b1):
    return pl.pallas_call(
        _table_body,
        grid=(NPAIR,),
        in_specs=[
            pl.BlockSpec((VPAD, EMB), lambda s: (0, 0)),
            pl.BlockSpec((1, EMB, HID), lambda s: (s, 0, 0)),
            pl.BlockSpec((1, EMB, HID), lambda s: (s + NPAIR, 0, 0)),
            pl.BlockSpec((1, HID), lambda s: (0, 0)),
        ],
        out_specs=pl.BlockSpec((1, VPAD, VPAD, HID), lambda s: (s, 0, 0, 0)),
        out_shape=jax.ShapeDtypeStruct((NPAIR, VPAD, VPAD, HID), jnp.float32),
    )(emb, w1r, w1r, b1.reshape(1, HID))


@functools.cache
def _make_sc_forward():
    mesh = plsc.VectorSubcoreMesh(core_axis_name="c", subcore_axis_name="s")
    return functools.partial(
        pl.kernel,
        mesh=mesh,
        out_type=jax.ShapeDtypeStruct((BATCH,), jnp.float32),
        scratch_types=[
            pltpu.VMEM((BPW * SEQ,), jnp.int32),
            pltpu.VMEM((2, 4 * NPAIR), jnp.int32),
            pltpu.VMEM((2, 4 * NPAIR, HID), jnp.float32),
            pltpu.VMEM((HID,), jnp.float32),
            pltpu.VMEM((L,), jnp.float32),
            pltpu.VMEM((BPW,), jnp.float32),
            pltpu.SemaphoreType.DMA,
            pltpu.SemaphoreType.DMA,
        ],
    )(_sc_body)


_GDN = lax.GatherDimensionNumbers(
    offset_dims=(), collapsed_slice_dims=(0,), start_index_map=(0,))


def _lane_perm(v, idx):
    return lax.gather(v, idx[:, None], _GDN, slice_sizes=(1,),
                      mode=lax.GatherScatterMode.PROMISE_IN_BOUNDS)


def _lane_sum(v):
    lanes = lax.iota(jnp.int32, L)
    for k in (1, 2, 4, 8):
        v = v + _lane_perm(v, lanes ^ k)
    return v


def _sc_body(x_hbm, t2_hbm, w2_hbm, b2_hbm, out_hbm,
             xs_v, idx_v, rows_v, w2_v, b2_v, log_v, sem0, sem1):
    wid = lax.axis_index("s") * NC + lax.axis_index("c")
    base = wid * BPW

    pltpu.sync_copy(x_hbm.at[pl.ds(base * SEQ, BPW * SEQ)], xs_v)
    pltpu.sync_copy(w2_hbm, w2_v)
    pltpu.sync_copy(b2_hbm, b2_v)

    sems = (sem0, sem1)
    offs = [(lax.iota(jnp.int32, L) + k * L) * PROWS for k in range(NPAIR // L)]

    def compute_idx(i, b):
        for q in range(4):
            iq = i + q
            xv = [xs_v[pl.ds(iq * SEQ + k * L, L)] for k in range(SEQ // L)]
            for k in range(NPAIR // L):
                idx_v[b, pl.ds(q * NPAIR + k * L, L)] = (
                    offs[k] + xv[k] * VPAD + xv[k + NPAIR // L])

    def start(b):
        pltpu.async_copy(t2_hbm.at[idx_v.at[b]], rows_v.at[b], sems[b])

    def wait(b):
        pltpu.make_async_copy(t2_hbm.at[idx_v.at[b]], rows_v.at[b],
                              sems[b]).wait()

    lanes = lax.iota(jnp.int32, L)

    def accumulate(i, q, b, logacc):
        def chunk_body(c, dot):
            co = c * L
            acc = rows_v[b, q * NPAIR, pl.ds(co, L)]
            for j in range(1, NPAIR):
                acc = acc + rows_v[b, q * NPAIR + j, pl.ds(co, L)]
            h = jnp.maximum(acc, 0.0)
            return dot + h * w2_v[pl.ds(co, L)]

        dot = lax.fori_loop(0, HID // L, chunk_body, b2_v[...])
        tot = _lane_sum(dot)
        return logacc + jnp.where(lanes == ((i + q) & (L - 1)), tot, 0.0)

    def accumulate4(i, b, logacc):
        for q in range(4):
            logacc = accumulate(i, q, b, logacc)
        return logacc

    compute_idx(0, 0)
    start(0)

    def pair_body(p, logacc):
        i0 = 8 * p
        compute_idx(i0 + 4, 1)
        start(1)
        wait(0)
        logacc = accumulate4(i0, 0, logacc)

        @pl.when(p < BPW // 8 - 1)
        def _():
            compute_idx(i0 + 8, 0)
            start(0)

        wait(1)
        logacc = accumulate4(i0 + 4, 1, logacc)

        group_done = (p & 1) == 1

        @pl.when(group_done)
        def _():
            log_v[pl.ds((p >> 1) * L, L)] = logacc

        return jnp.where(group_done, 0.0, logacc)

    lax.fori_loop(0, BPW // 8, pair_body, jnp.zeros((L,), jnp.float32))

    pltpu.sync_copy(log_v, out_hbm.at[pl.ds(base, BPW)])


def kernel(batch_x, emb_table, W1, b1, W2, b2):
    w1r = W1.reshape(SEQ, EMB, HID)
    emb_pad = jnp.zeros((VPAD, EMB), jnp.float32).at[:VOCAB].set(emb_table)
    table = _build_table(emb_pad, w1r, b1).reshape(NROWS, HID)

    x_flat = batch_x.astype(jnp.int32).reshape(-1)
    w2 = W2.reshape(HID)
    b2_pad = jnp.zeros((L,), jnp.float32).at[0].set(b2[0])
    return _make_sc_forward()(x_flat, table, w2, b2_pad)
